# baseline (speedup 1.0000x reference)
import functools

import jax
import jax.numpy as jnp
from jax import lax
from jax.experimental import pallas as pl
from jax.experimental.pallas import tpu as pltpu
from jax.experimental.pallas import tpu_sc as plsc

_LANES = 512


def _copy_swap_body(t_ref, mem_ref, win_ref, idx_ref, ind_ref, omem_ref, oidx_ref):
    i = pl.program_id(0)
    t = t_ref[0]

    @pl.when(i == t)
    def _():
        omem_ref[...] = win_ref[...]
        oidx_ref[...] = ind_ref[...]

    @pl.when(i != t)
    def _():
        omem_ref[...] = mem_ref[...]
        oidx_ref[...] = idx_ref[...]


def _queue_update(mem2d, win2d, idx2d, ind2d, t):
    rw = win2d.shape[0]
    ir = ind2d.shape[0]
    n = mem2d.shape[0] // rw
    return pl.pallas_call(
        _copy_swap_body,
        grid=(n,),
        in_specs=[
            pl.BlockSpec(memory_space=pltpu.SMEM),
            pl.BlockSpec((rw, _LANES), lambda i: (i, 0)),
            pl.BlockSpec((rw, _LANES), lambda i: (0, 0)),
            pl.BlockSpec((ir, _LANES), lambda i: (i, 0)),
            pl.BlockSpec((ir, _LANES), lambda i: (0, 0)),
        ],
        out_specs=[
            pl.BlockSpec((rw, _LANES), lambda i: (i, 0)),
            pl.BlockSpec((ir, _LANES), lambda i: (i, 0)),
        ],
        out_shape=[
            jax.ShapeDtypeStruct(mem2d.shape, mem2d.dtype),
            jax.ShapeDtypeStruct(idx2d.shape, idx2d.dtype),
        ],
        compiler_params=pltpu.CompilerParams(dimension_semantics=("parallel",)),
    )(t, mem2d, win2d, idx2d, ind2d)


def _sc_gather_pairs(table2, idx2):
    info = plsc.get_sparse_core_info()
    nc, ns = info.num_cores, info.num_subcores
    nw = nc * ns
    b = idx2.shape[0]
    dw = table2.shape[1]
    bpw = b // nw
    mesh = plsc.VectorSubcoreMesh(core_axis_name="c", subcore_axis_name="s")

    @functools.partial(
        pl.kernel,
        out_type=jax.ShapeDtypeStruct((b, dw), table2.dtype),
        mesh=mesh,
        scratch_types=[
            pltpu.VMEM((bpw,), jnp.int32),
            pltpu.VMEM((bpw, dw), table2.dtype),
            pltpu.SemaphoreType.DMA,
        ],
    )
    def k(table_hbm, idx_hbm, out_hbm, idx_v, rows_v, sem):
        wid = lax.axis_index("s") * nc + lax.axis_index("c")
        base = wid * bpw
        pltpu.sync_copy(idx_hbm.at[pl.ds(base, bpw)], idx_v)
        pltpu.async_copy(table_hbm.at[idx_v], rows_v, sem).wait()
        pltpu.sync_copy(rows_v, out_hbm.at[pl.ds(base, bpw)])

    return k(table2, idx2)


def _half_select_body(pairs_ref, par_ref, o_ref):
    d = o_ref.shape[1]
    take_left = par_ref[...] == 0
    o_ref[...] = jnp.where(take_left, pairs_ref[:, :d], pairs_ref[:, d:])


def _half_select(pairs, parity_col):
    b, dw = pairs.shape
    d = dw // 2
    return pl.pallas_call(
        _half_select_body,
        out_shape=jax.ShapeDtypeStruct((b, d), pairs.dtype),
    )(pairs, parity_col)


def kernel(mem_ref, mem_pos, Y_ref, Z, indices, idx_ref_buf, idx_pos_buf,
           pos_sampled_idx, step_rel):
    B, d = Y_ref.shape
    R = mem_ref.shape[0]
    P = mem_pos.shape[0]

    step = jnp.asarray(step_rel, jnp.int32)
    t_ref = jnp.reshape(((step * B) % R) // B, (1,))
    t_pos = jnp.reshape(((step * B) % P) // B, (1,))

    mem_ref2d = mem_ref.reshape(-1, _LANES)
    y2d = Y_ref.reshape(-1, _LANES)
    idx_ref2d = idx_ref_buf.reshape(-1, _LANES)
    mem_pos2d = mem_pos.reshape(-1, _LANES)
    z2d = Z.reshape(-1, _LANES)
    idx_pos2d = idx_pos_buf.reshape(-1, _LANES)
    ind2d = indices.reshape(-1, _LANES)

    mem_pos_new2d, idx_pos_new2d = _queue_update(mem_pos2d, z2d, idx_pos2d, ind2d, t_pos)
    mem_pos_new = mem_pos_new2d.reshape(P, d)
    idx_pos_new = idx_pos_new2d.reshape(P)

    mem_ref_new2d, idx_ref_new2d = _queue_update(mem_ref2d, y2d, idx_ref2d, ind2d, t_ref)

    idx2 = jax.lax.shift_right_logical(pos_sampled_idx, 1)
    parity_col = jnp.bitwise_and(pos_sampled_idx, 1).reshape(B, 1)
    pairs = _sc_gather_pairs(mem_pos_new.reshape(P // 2, 2 * d), idx2)
    Z_pseudo = _half_select(pairs, parity_col)

    return (mem_ref_new2d.reshape(R, d), idx_ref_new2d.reshape(R),
            mem_pos_new, idx_pos_new, Z_pseudo)


# speedup vs baseline: 1.3497x; 1.3497x over previous
import functools

import jax
import jax.numpy as jnp
from jax import lax
from jax.experimental import pallas as pl
from jax.experimental.pallas import tpu as pltpu
from jax.experimental.pallas import tpu_sc as plsc


def _copy_swap_body(t_ref, mem_ref, win_ref, idx_ref, ind_ref, omem_ref, oidx_ref):
    i = pl.program_id(0)
    t = t_ref[0]

    @pl.when(i == t)
    def _():
        omem_ref[...] = win_ref[...]
        oidx_ref[...] = ind_ref[...]

    @pl.when(i != t)
    def _():
        omem_ref[...] = mem_ref[...]
        oidx_ref[...] = idx_ref[...]


def _queue_update(mem, win, idx, ind, t):
    b, d = win.shape
    n = mem.shape[0] // b
    return pl.pallas_call(
        _copy_swap_body,
        grid=(n,),
        in_specs=[
            pl.BlockSpec(memory_space=pltpu.SMEM),
            pl.BlockSpec((b, d), lambda i: (i, 0)),
            pl.BlockSpec((b, d), lambda i: (0, 0)),
            pl.BlockSpec((b,), lambda i: (i,)),
            pl.BlockSpec((b,), lambda i: (0,)),
        ],
        out_specs=[
            pl.BlockSpec((b, d), lambda i: (i, 0)),
            pl.BlockSpec((b,), lambda i: (i,)),
        ],
        out_shape=[
            jax.ShapeDtypeStruct(mem.shape, mem.dtype),
            jax.ShapeDtypeStruct(idx.shape, idx.dtype),
        ],
        compiler_params=pltpu.CompilerParams(dimension_semantics=("parallel",)),
    )(t, mem, win, idx, ind)


def _sc_gather(table, idx):
    info = plsc.get_sparse_core_info()
    nc = info.num_cores
    b = idx.shape[0]
    d = table.shape[1]
    bpw = b // nc
    mesh = plsc.ScalarSubcoreMesh(axis_name="core", num_cores=nc)

    @functools.partial(
        pl.kernel,
        out_type=jax.ShapeDtypeStruct((b, d), table.dtype),
        mesh=mesh,
        scratch_types=[
            pltpu.SMEM((bpw,), jnp.int32),
            pltpu.SemaphoreType.DMA,
            pltpu.SemaphoreType.DMA,
        ],
    )
    def k(table_hbm, idx_hbm, out_hbm, idx_s, sem_i, sem):
        cid = lax.axis_index("core")
        base = cid * bpw
        pltpu.async_copy(idx_hbm.at[pl.ds(base, bpw)], idx_s, sem_i).wait()

        @pl.loop(0, bpw)
        def _(r):
            j = idx_s[r]
            pltpu.make_async_copy(
                table_hbm.at[pl.ds(j, 1), :],
                out_hbm.at[pl.ds(base + r, 1), :],
                sem,
            ).start()

        pltpu.make_async_copy(
            table_hbm.at[pl.ds(0, bpw), :],
            out_hbm.at[pl.ds(base, bpw), :],
            sem,
        ).wait()

    return k(table, idx)


def kernel(mem_ref, mem_pos, Y_ref, Z, indices, idx_ref_buf, idx_pos_buf,
           pos_sampled_idx, step_rel):
    B, d = Y_ref.shape
    R = mem_ref.shape[0]
    P = mem_pos.shape[0]

    step = jnp.asarray(step_rel, jnp.int32)
    t_ref = jnp.reshape(((step * B) % R) // B, (1,))
    t_pos = jnp.reshape(((step * B) % P) // B, (1,))

    mem_pos_new, idx_pos_new = _queue_update(mem_pos, Z, idx_pos_buf, indices, t_pos)

    mem_ref_new, idx_ref_new = _queue_update(mem_ref, Y_ref, idx_ref_buf, indices, t_ref)

    Z_pseudo = _sc_gather(mem_pos_new, pos_sampled_idx)

    return mem_ref_new, idx_ref_new, mem_pos_new, idx_pos_new, Z_pseudo
